# baseline (speedup 1.0000x reference)
import functools

import jax
import jax.numpy as jnp
from jax import lax
from jax.experimental import pallas as pl
from jax.experimental.pallas import tpu as pltpu
from jax.experimental.pallas import tpu_sc as plsc

VOCAB = 16
D = 16
IDS_PER_ROW = 128
KROWS = 8
CHUNK = KROWS * IDS_PER_ROW


def _h_body(e_ref, w_ref, b_ref, h_ref, g_ref):
    h = lax.dot_general(
        e_ref[...], w_ref[...],
        (((1,), (1,)), ((), ())),
        preferred_element_type=jnp.float32,
    )
    h = h + b_ref[...]
    h_ref[...] = h
    g_ref[...] = jnp.broadcast_to(h[:, 0:1], (VOCAB, D))


def _compute_tables(embed_table, W, b):
    b_mat = jnp.broadcast_to(b.reshape(1, D), (VOCAB, D))
    return pl.pallas_call(
        _h_body,
        out_shape=[
            jax.ShapeDtypeStruct((VOCAB, D), jnp.float32),
            jax.ShapeDtypeStruct((VOCAB, D), jnp.float32),
        ],
    )(embed_table, W, b_mat)


def _sc_gather(ids2d, h_tab, g_tab):
    n_rows = ids2d.shape[0]
    n = n_rows * IDS_PER_ROW
    info = plsc.get_sparse_core_info()
    nc, ns = info.num_cores, info.num_subcores
    nw = nc * ns
    rows_per_w = n_rows // nw
    assert rows_per_w * nw == n_rows and rows_per_w % KROWS == 0
    n_chunks = rows_per_w // KROWS
    assert n_chunks % 2 == 0
    n_iter = n_chunks // 2

    mesh = plsc.VectorSubcoreMesh(core_axis_name="c", subcore_axis_name="s")

    @functools.partial(
        pl.kernel,
        out_type=[
            jax.ShapeDtypeStruct((n, D), jnp.float32),
            jax.ShapeDtypeStruct((n, D), jnp.float32),
        ],
        mesh=mesh,
        scratch_types=[
            pltpu.VMEM((KROWS, IDS_PER_ROW), jnp.int32),
            pltpu.VMEM((KROWS, IDS_PER_ROW), jnp.int32),
            pltpu.VMEM((CHUNK, D), jnp.float32),
            pltpu.VMEM((CHUNK, D), jnp.float32),
            pltpu.VMEM((CHUNK, D), jnp.float32),
            pltpu.VMEM((CHUNK, D), jnp.float32),
            pltpu.SemaphoreType.DMA,
            pltpu.SemaphoreType.DMA,
            pltpu.SemaphoreType.DMA,
            pltpu.SemaphoreType.DMA,
            pltpu.SemaphoreType.DMA,
        ],
        compiler_params=pltpu.CompilerParams(
            needs_layout_passes=False, use_tc_tiling_on_sc=False),
    )
    def k(ids_hbm, h_hbm, g_hbm, hid_hbm, log_hbm,
          iv0, iv1, hv0, hv1, lv0, lv1, si0, si1, sg, sw0, sw1):
        wid = lax.axis_index("s") * nc + lax.axis_index("c")
        row_base = wid * rows_per_w
        idbufs = (iv0, iv1)
        hbufs = (hv0, hv1)
        lbufs = (lv0, lv1)
        isems = (si0, si1)
        wsems = (sw0, sw1)

        def ids_src(c):
            return ids_hbm.at[pl.ds(row_base + c * KROWS, KROWS)]

        def out_rows(hbm, c):
            return hbm.at[pl.ds((row_base + c * KROWS) * IDS_PER_ROW, CHUNK)]

        pltpu.async_copy(ids_src(0), iv0, si0)

        def iter_body(i, carry):
            for b in range(2):
                c = i * 2 + b
                pltpu.make_async_copy(ids_src(c), idbufs[b], isems[b]).wait()

                @pl.when(i >= 1)
                def _():
                    pltpu.make_async_copy(
                        hbufs[b], out_rows(hid_hbm, c), wsems[b]).wait()
                    pltpu.make_async_copy(
                        lbufs[b], out_rows(log_hbm, c), wsems[b]).wait()

                cps = []
                for j in range(KROWS):
                    dst = pl.ds(j * IDS_PER_ROW, IDS_PER_ROW)
                    cps.append(pltpu.async_copy(
                        h_hbm.at[idbufs[b].at[j]], hbufs[b].at[dst], sg))
                    cps.append(pltpu.async_copy(
                        g_hbm.at[idbufs[b].at[j]], lbufs[b].at[dst], sg))

                if b == 0:
                    pltpu.async_copy(ids_src(c + 1), idbufs[1], isems[1])
                else:
                    @pl.when(i < n_iter - 1)
                    def _():
                        pltpu.async_copy(ids_src(c + 1), idbufs[0], isems[0])

                for cp in cps:
                    cp.wait()
                pltpu.async_copy(hbufs[b], out_rows(hid_hbm, c), wsems[b])
                pltpu.async_copy(lbufs[b], out_rows(log_hbm, c), wsems[b])
            return carry

        lax.fori_loop(0, n_iter, iter_body, 0, unroll=False)

        for b in range(2):
            c = n_chunks - 2 + b
            pltpu.make_async_copy(
                hbufs[b], out_rows(hid_hbm, c), wsems[b]).wait()
            pltpu.make_async_copy(
                lbufs[b], out_rows(log_hbm, c), wsems[b]).wait()

    return k(ids2d, h_tab, g_tab)


def kernel(input_ids, embed_table, W, b):
    bsz, seq = input_ids.shape
    ids2d = input_ids.reshape(-1, IDS_PER_ROW).astype(jnp.int32)
    h_tab, g_tab = _compute_tables(embed_table, W, b)
    hid_flat, log_flat = _sc_gather(ids2d, h_tab, g_tab)
    hidden = hid_flat.reshape(bsz, seq, D)
    logits = log_flat.reshape(bsz, seq, D)
    return (logits, hidden)


# speedup vs baseline: 2.4629x; 2.4629x over previous
import functools

import jax
import jax.numpy as jnp
from jax import lax
from jax.experimental import pallas as pl
from jax.experimental.pallas import tpu as pltpu
from jax.experimental.pallas import tpu_sc as plsc

VOCAB = 16
D = 16
CHUNK = 1024
GROUPS = CHUNK // 16


def _h_body(e_ref, w_ref, b_ref, h_ref):
    h = lax.dot_general(
        e_ref[...], w_ref[...],
        (((1,), (1,)), ((), ())),
        preferred_element_type=jnp.float32,
    )
    h_ref[...] = h + b_ref[...]


def _compute_h(embed_table, W, b):
    b_mat = jnp.broadcast_to(b.reshape(1, D), (VOCAB, D))
    return pl.pallas_call(
        _h_body,
        out_shape=jax.ShapeDtypeStruct((VOCAB, D), jnp.float32),
    )(embed_table, W, b_mat)


def _sc_gather(ids, h_flat):
    n = ids.shape[0]
    info = plsc.get_sparse_core_info()
    nc, ns = info.num_cores, info.num_subcores
    nw = nc * ns
    per_w = n // nw
    assert per_w * nw == n and per_w % CHUNK == 0
    n_chunks = per_w // CHUNK
    assert n_chunks % 2 == 0
    n_iter = n_chunks // 2

    mesh = plsc.VectorSubcoreMesh(core_axis_name="c", subcore_axis_name="s")

    @functools.partial(
        pl.kernel,
        out_type=[
            jax.ShapeDtypeStruct((n * D,), jnp.float32),
            jax.ShapeDtypeStruct((n * D,), jnp.float32),
        ],
        mesh=mesh,
        scratch_types=[
            pltpu.VMEM((VOCAB * D,), jnp.float32),
            pltpu.VMEM((CHUNK,), jnp.int32),
            pltpu.VMEM((CHUNK,), jnp.int32),
            pltpu.VMEM((CHUNK * D,), jnp.float32),
            pltpu.VMEM((CHUNK * D,), jnp.float32),
            pltpu.VMEM((CHUNK * D,), jnp.float32),
            pltpu.VMEM((CHUNK * D,), jnp.float32),
            pltpu.SemaphoreType.DMA,
            pltpu.SemaphoreType.DMA,
            pltpu.SemaphoreType.DMA,
            pltpu.SemaphoreType.DMA,
        ],
        compiler_params=pltpu.CompilerParams(
            needs_layout_passes=False, use_tc_tiling_on_sc=False),
    )
    def k(ids_hbm, h_hbm, hid_hbm, log_hbm,
          h_v, iv0, iv1, hv0, hv1, lv0, lv1, si0, si1, sw0, sw1):
        wid = lax.axis_index("s") * nc + lax.axis_index("c")
        base = wid * per_w
        idbufs = (iv0, iv1)
        hbufs = (hv0, hv1)
        lbufs = (lv0, lv1)
        isems = (si0, si1)
        wsems = (sw0, sw1)

        pltpu.sync_copy(h_hbm, h_v)
        lane16 = lax.iota(jnp.int32, 16) * D

        def ids_src(c):
            return ids_hbm.at[pl.ds(base + c * CHUNK, CHUNK)]

        def out_dst(hbm, c):
            return hbm.at[pl.ds((base + c * CHUNK) * D, CHUNK * D)]

        pltpu.async_copy(ids_src(0), iv0, si0)

        def iter_body(i, carry):
            for b in range(2):
                c = i * 2 + b
                pltpu.make_async_copy(ids_src(c), idbufs[b], isems[b]).wait()

                if b == 0:
                    pltpu.async_copy(ids_src(c + 1), idbufs[1], isems[1])
                else:
                    @pl.when(i < n_iter - 1)
                    def _():
                        pltpu.async_copy(ids_src(c + 1), idbufs[0], isems[0])

                @pl.when(i >= 1)
                def _():
                    pltpu.make_async_copy(
                        hbufs[b], out_dst(hid_hbm, c), wsems[b]).wait()
                    pltpu.make_async_copy(
                        lbufs[b], out_dst(log_hbm, c), wsems[b]).wait()

                hid_v, log_v, ids_v = hbufs[b], lbufs[b], idbufs[b]

                def group_body(g, carry2):
                    idv = ids_v[pl.ds(g * 16, 16)]
                    bi = idv * D
                    g0 = plsc.load_gather(h_v, (bi,))
                    pos0 = lane16 + g * (16 * D)
                    for j in range(D):
                        r = plsc.load_gather(h_v, (bi + j,))
                        plsc.store_scatter(hid_v, (pos0 + j,), r)
                        plsc.store_scatter(log_v, (pos0 + j,), g0)
                    return carry2

                lax.fori_loop(0, GROUPS, group_body, 0, unroll=4)

                pltpu.async_copy(hid_v, out_dst(hid_hbm, c), wsems[b])
                pltpu.async_copy(log_v, out_dst(log_hbm, c), wsems[b])
            return carry

        lax.fori_loop(0, n_iter, iter_body, 0, unroll=False)

        for b in range(2):
            c = n_chunks - 2 + b
            pltpu.make_async_copy(
                hbufs[b], out_dst(hid_hbm, c), wsems[b]).wait()
            pltpu.make_async_copy(
                lbufs[b], out_dst(log_hbm, c), wsems[b]).wait()

    return k(ids, h_flat)


def kernel(input_ids, embed_table, W, b):
    bsz, seq = input_ids.shape
    ids = input_ids.reshape(-1).astype(jnp.int32)
    h = _compute_h(embed_table, W, b)
    hid_flat, log_flat = _sc_gather(ids, h.reshape(-1))
    hidden = hid_flat.reshape(bsz, seq, D)
    logits = log_flat.reshape(bsz, seq, D)
    return (logits, hidden)
